# Initial kernel scaffold; baseline (speedup 1.0000x reference)
import jax
import jax.numpy as jnp
from jax.experimental import pallas as pl


def kernel(x, edge_index, batch, W0, b0, W1, b1):
    raise NotImplementedError("write your pallas kernel here")


# speedup vs baseline: 21.7111x; 21.7111x over previous
import functools

import jax
import jax.numpy as jnp
from jax import lax
from jax.experimental import pallas as pl
from jax.experimental.pallas import tpu as pltpu
from jax.experimental.pallas import tpu_sc as plsc

N = 10000
E = 320000
D = 128
G = 64

NC = 2
NS = 16
NW = NC * NS
EPT = E // NW
CH = 125
NCH = EPT // CH
RCH = 80
NRCH = N // RCH

BN = 400
NB = N // BN

_mesh = plsc.VectorSubcoreMesh(core_axis_name="c", subcore_axis_name="s")
_sc_params = pltpu.CompilerParams(needs_layout_passes=False)


def _deg_body(dst_hbm, out_hbm, didx, deg_v):
    c = lax.axis_index("c")
    s = lax.axis_index("s")
    wid = c * NS + s

    pltpu.sync_copy(dst_hbm.at[pl.ds(wid * EPT, EPT)], didx)

    zeros16 = jnp.zeros((16,), jnp.float32)

    def _zero(i, carry):
        deg_v[pl.ds(i * 16, 16)] = zeros16
        return carry

    lax.fori_loop(0, N // 16, _zero, 0)

    ones16 = jnp.ones((16,), jnp.float32)

    def _count(i, carry):
        idx = didx[pl.ds(i * 16, 16)]
        plsc.addupdate_scatter(deg_v, [idx], ones16)
        return carry

    lax.fori_loop(0, EPT // 16, _count, 0)

    pltpu.sync_copy(deg_v, out_hbm.at[wid])


_deg = functools.partial(
    pl.kernel,
    out_type=jax.ShapeDtypeStruct((NW, N), jnp.float32),
    mesh=_mesh,
    compiler_params=_sc_params,
    scratch_types=[
        pltpu.VMEM((EPT,), jnp.int32),
        pltpu.VMEM((N,), jnp.float32),
    ],
)(_deg_body)


def _agg_body(hs_hbm, src_hbm, dst_hbm, zero_hbm, out_hbm, sidx, didx, rows, acc):
    c = lax.axis_index("c")
    s = lax.axis_index("s")
    wid = c * NS + s

    pltpu.sync_copy(src_hbm.at[wid], sidx)
    pltpu.sync_copy(dst_hbm.at[wid], didx)

    pltpu.sync_copy(zero_hbm, rows.at[pl.ds(0, RCH)])
    nk = (NRCH - s + NS - 1) // NS

    def _zacc(k, carry):
        off = (s + k * NS) * RCH
        pltpu.sync_copy(rows.at[pl.ds(0, RCH)], acc.at[pl.ds(off, RCH)])
        return carry

    lax.fori_loop(0, nk, _zacc, 0)
    plsc.subcore_barrier()

    def _step(i, carry):
        pltpu.sync_copy(hs_hbm.at[sidx.at[i]], rows)
        pltpu.sync_copy(rows, acc.at[didx.at[i]], add=True)
        return carry

    lax.fori_loop(0, NCH, _step, 0)
    plsc.subcore_barrier()

    def _wb(k, carry):
        off = (s + k * NS) * RCH
        pltpu.sync_copy(acc.at[pl.ds(off, RCH)], rows.at[pl.ds(0, RCH)])
        pltpu.sync_copy(rows.at[pl.ds(0, RCH)], out_hbm.at[c, pl.ds(off, RCH)])
        return carry

    lax.fori_loop(0, nk, _wb, 0)


_agg = functools.partial(
    pl.kernel,
    out_type=jax.ShapeDtypeStruct((NC, N, D), jnp.float32),
    mesh=_mesh,
    compiler_params=_sc_params,
    scratch_types=[
        pltpu.VMEM((NCH, CH), jnp.int32),
        pltpu.VMEM((NCH, CH), jnp.int32),
        pltpu.VMEM((CH, D), jnp.float32),
        pltpu.VMEM_SHARED((N, D), jnp.float32),
    ],
)(_agg_body)


def _dis_block(deg_ref):
    deg = jnp.sum(deg_ref[...], axis=1, keepdims=True) + 1.0
    return lax.rsqrt(deg)


def _tc1_body(deg_ref, x_ref, w_ref, hs_ref):
    dis = _dis_block(deg_ref)
    h = jnp.dot(x_ref[...], w_ref[...], precision=lax.Precision.HIGHEST,
                preferred_element_type=jnp.float32)
    hs_ref[...] = h * dis


_tc1 = pl.pallas_call(
    _tc1_body,
    grid=(NB,),
    in_specs=[
        pl.BlockSpec((BN, NW), lambda i: (i, 0)),
        pl.BlockSpec((BN, D), lambda i: (i, 0)),
        pl.BlockSpec((D, D), lambda i: (0, 0)),
    ],
    out_specs=pl.BlockSpec((BN, D), lambda i: (i, 0)),
    out_shape=jax.ShapeDtypeStruct((N, D), jnp.float32),
)


def _tc2_body(acc_ref, hs_ref, deg_ref, b_ref, w_ref, out_ref):
    dis = _dis_block(deg_ref)
    t = (acc_ref[0] + acc_ref[1] + hs_ref[...]) * dis + b_ref[...]
    h1 = jnp.maximum(t, 0.0)
    out_ref[...] = jnp.dot(h1, w_ref[...], precision=lax.Precision.HIGHEST,
                           preferred_element_type=jnp.float32) * dis


_tc2 = pl.pallas_call(
    _tc2_body,
    grid=(NB,),
    in_specs=[
        pl.BlockSpec((NC, BN, D), lambda i: (0, i, 0)),
        pl.BlockSpec((BN, D), lambda i: (i, 0)),
        pl.BlockSpec((BN, NW), lambda i: (i, 0)),
        pl.BlockSpec((1, D), lambda i: (0, 0)),
        pl.BlockSpec((D, D), lambda i: (0, 0)),
    ],
    out_specs=pl.BlockSpec((BN, D), lambda i: (i, 0)),
    out_shape=jax.ShapeDtypeStruct((N, D), jnp.float32),
)


def _tc3_body(acc_ref, hs_ref, deg_ref, b_ref, batch_ref, h2_ref, pooled_ref,
              s_acc, c_acc):
    i = pl.program_id(0)
    dis = _dis_block(deg_ref)
    t = (acc_ref[0] + acc_ref[1] + hs_ref[...]) * dis + b_ref[...]
    h2 = jnp.maximum(t, 0.0)
    h2_ref[...] = h2

    bt = batch_ref[...].reshape(1, BN)
    gid = lax.broadcasted_iota(jnp.int32, (G, BN), 0)
    oh = jnp.where(gid == bt, 1.0, 0.0).astype(jnp.float32)
    sblk = jnp.dot(oh, h2, precision=lax.Precision.HIGHEST,
                   preferred_element_type=jnp.float32)
    cblk = jnp.broadcast_to(jnp.sum(oh, axis=1, keepdims=True), (G, D))

    @pl.when(i == 0)
    def _():
        s_acc[...] = sblk
        c_acc[...] = cblk

    @pl.when(i > 0)
    def _():
        s_acc[...] += sblk
        c_acc[...] += cblk

    @pl.when(i == NB - 1)
    def _():
        pooled_ref[...] = s_acc[...] / jnp.maximum(c_acc[...], 1.0)


_tc3 = pl.pallas_call(
    _tc3_body,
    grid=(NB,),
    in_specs=[
        pl.BlockSpec((NC, BN, D), lambda i: (0, i, 0)),
        pl.BlockSpec((BN, D), lambda i: (i, 0)),
        pl.BlockSpec((BN, NW), lambda i: (i, 0)),
        pl.BlockSpec((1, D), lambda i: (0, 0)),
        pl.BlockSpec((1, 1, BN), lambda i: (i, 0, 0)),
    ],
    out_specs=[
        pl.BlockSpec((BN, D), lambda i: (i, 0)),
        pl.BlockSpec((G, D), lambda i: (0, 0)),
    ],
    out_shape=[
        jax.ShapeDtypeStruct((N, D), jnp.float32),
        jax.ShapeDtypeStruct((G, D), jnp.float32),
    ],
    scratch_shapes=[
        pltpu.VMEM((G, D), jnp.float32),
        pltpu.VMEM((G, D), jnp.float32),
    ],
)


def kernel(x, edge_index, batch, W0, b0, W1, b1):
    src3 = edge_index[0].reshape(NW, NCH, CH)
    dst3 = edge_index[1].reshape(NW, NCH, CH)
    dst1 = edge_index[1]
    zero_rows = jnp.zeros((RCH, D), jnp.float32)

    degp = _deg(dst1)
    deg_t = degp.T

    hs0 = _tc1(deg_t, x, W0)
    acc0 = _agg(hs0, src3, dst3, zero_rows)
    hs1 = _tc2(acc0, hs0, deg_t, b0.reshape(1, D), W1)
    acc1 = _agg(hs1, src3, dst3, zero_rows)
    h2, pooled = _tc3(acc1, hs1, deg_t, b1.reshape(1, D),
                      batch.reshape(NB, 1, BN))
    return (pooled, h2)


# speedup vs baseline: 29.5875x; 1.3628x over previous
import functools

import jax
import jax.numpy as jnp
from jax import lax
from jax.experimental import pallas as pl
from jax.experimental.pallas import tpu as pltpu
from jax.experimental.pallas import tpu_sc as plsc

N = 10000
E = 320000
D = 128
G = 64

NC = 2
NS = 16
NW = NC * NS
EPT = E // NW
CH = 100
NCH = EPT // CH
NSTG = 2
PCH = NCH // NSTG
RCH = 80
NRCH = N // RCH

BN = 400
NB = N // BN

_mesh = plsc.VectorSubcoreMesh(core_axis_name="c", subcore_axis_name="s")
_sc_params = pltpu.CompilerParams(needs_layout_passes=False)


def _deg_body(dst_hbm, out_hbm, didx, deg_v):
    c = lax.axis_index("c")
    s = lax.axis_index("s")
    wid = c * NS + s

    pltpu.sync_copy(dst_hbm.at[pl.ds(wid * EPT, EPT)], didx)

    zeros16 = jnp.zeros((16,), jnp.float32)

    def _zero(i, carry):
        deg_v[pl.ds(i * 16, 16)] = zeros16
        return carry

    lax.fori_loop(0, N // 16, _zero, 0)

    ones16 = jnp.ones((16,), jnp.float32)

    def _count(i, carry):
        idx = didx[pl.ds(i * 16, 16)]
        plsc.addupdate_scatter(deg_v, [idx], ones16)
        return carry

    lax.fori_loop(0, EPT // 16, _count, 0)

    pltpu.sync_copy(deg_v, out_hbm.at[wid])


_deg = functools.partial(
    pl.kernel,
    out_type=jax.ShapeDtypeStruct((NW, N), jnp.float32),
    mesh=_mesh,
    compiler_params=_sc_params,
    scratch_types=[
        pltpu.VMEM((EPT,), jnp.int32),
        pltpu.VMEM((N,), jnp.float32),
    ],
)(_deg_body)


NBUF = 2


def _agg_body(hs_hbm, src_hbm, dst_hbm, zero_hbm, out_hbm, sidx, didx,
              rowsbuf, acc, sem0, sem1):
    c = lax.axis_index("c")
    s = lax.axis_index("s")
    wid = c * NS + s
    rows0 = rowsbuf.at[pl.ds(0, CH)]
    rows1 = rowsbuf.at[pl.ds(CH, CH)]
    rows = (rows0, rows1)
    sems = (sem0, sem1)

    pltpu.sync_copy(zero_hbm, rows0.at[pl.ds(0, RCH)])
    nk = (NRCH - s + NS - 1) // NS

    def _zacc(k, carry):
        off = (s + k * NS) * RCH
        pltpu.sync_copy(rows0.at[pl.ds(0, RCH)], acc.at[pl.ds(off, RCH)])
        return carry

    lax.fori_loop(0, nk, _zacc, 0)
    plsc.subcore_barrier()

    def _phase(p, carry):
        pltpu.sync_copy(src_hbm.at[wid, p], sidx)
        pltpu.sync_copy(dst_hbm.at[wid, p], didx)
        for b in range(NBUF):
            pltpu.async_copy(hs_hbm.at[sidx.at[b]], rows[b], sems[b])

        def _step(j, carry2):
            for b in range(NBUF):
                i = j * NBUF + b
                pltpu.make_async_copy(hs_hbm.at[sidx.at[i]], rows[b],
                                      sems[b]).wait()
                pltpu.sync_copy(rows[b], acc.at[didx.at[i]], add=True)

                @pl.when(j < PCH // NBUF - 1)
                def _():
                    pltpu.async_copy(hs_hbm.at[sidx.at[i + NBUF]], rows[b],
                                     sems[b])

            return carry2

        lax.fori_loop(0, PCH // NBUF, _step, 0)
        return carry

    lax.fori_loop(0, NSTG, _phase, 0)
    plsc.subcore_barrier()

    def _wb(k, carry):
        off = (s + k * NS) * RCH
        pltpu.sync_copy(acc.at[pl.ds(off, RCH)], rows0.at[pl.ds(0, RCH)])
        pltpu.sync_copy(rows0.at[pl.ds(0, RCH)], out_hbm.at[c, pl.ds(off, RCH)])
        return carry

    lax.fori_loop(0, nk, _wb, 0)


_agg = functools.partial(
    pl.kernel,
    out_type=jax.ShapeDtypeStruct((NC, N, D), jnp.float32),
    mesh=_mesh,
    compiler_params=_sc_params,
    scratch_types=[
        pltpu.VMEM((PCH, CH), jnp.int32),
        pltpu.VMEM((PCH, CH), jnp.int32),
        pltpu.VMEM((NBUF * CH, D), jnp.float32),
        pltpu.VMEM_SHARED((N, D), jnp.float32),
        pltpu.SemaphoreType.DMA,
        pltpu.SemaphoreType.DMA,
    ],
)(_agg_body)


def _dis_block(deg_ref):
    deg = jnp.sum(deg_ref[...], axis=1, keepdims=True) + 1.0
    return lax.rsqrt(deg)


def _tc1_body(deg_ref, x_ref, w_ref, hs_ref):
    dis = _dis_block(deg_ref)
    h = jnp.dot(x_ref[...], w_ref[...], precision=lax.Precision.HIGHEST,
                preferred_element_type=jnp.float32)
    hs_ref[...] = h * dis


_tc1 = pl.pallas_call(
    _tc1_body,
    grid=(NB,),
    in_specs=[
        pl.BlockSpec((BN, NW), lambda i: (i, 0)),
        pl.BlockSpec((BN, D), lambda i: (i, 0)),
        pl.BlockSpec((D, D), lambda i: (0, 0)),
    ],
    out_specs=pl.BlockSpec((BN, D), lambda i: (i, 0)),
    out_shape=jax.ShapeDtypeStruct((N, D), jnp.float32),
)


def _tc2_body(acc_ref, hs_ref, deg_ref, b_ref, w_ref, out_ref):
    dis = _dis_block(deg_ref)
    t = (acc_ref[0] + acc_ref[1] + hs_ref[...]) * dis + b_ref[...]
    h1 = jnp.maximum(t, 0.0)
    out_ref[...] = jnp.dot(h1, w_ref[...], precision=lax.Precision.HIGHEST,
                           preferred_element_type=jnp.float32) * dis


_tc2 = pl.pallas_call(
    _tc2_body,
    grid=(NB,),
    in_specs=[
        pl.BlockSpec((NC, BN, D), lambda i: (0, i, 0)),
        pl.BlockSpec((BN, D), lambda i: (i, 0)),
        pl.BlockSpec((BN, NW), lambda i: (i, 0)),
        pl.BlockSpec((1, D), lambda i: (0, 0)),
        pl.BlockSpec((D, D), lambda i: (0, 0)),
    ],
    out_specs=pl.BlockSpec((BN, D), lambda i: (i, 0)),
    out_shape=jax.ShapeDtypeStruct((N, D), jnp.float32),
)


def _tc3_body(acc_ref, hs_ref, deg_ref, b_ref, batch_ref, h2_ref, pooled_ref,
              s_acc, c_acc):
    i = pl.program_id(0)
    dis = _dis_block(deg_ref)
    t = (acc_ref[0] + acc_ref[1] + hs_ref[...]) * dis + b_ref[...]
    h2 = jnp.maximum(t, 0.0)
    h2_ref[...] = h2

    bt = batch_ref[...].reshape(1, BN)
    gid = lax.broadcasted_iota(jnp.int32, (G, BN), 0)
    oh = jnp.where(gid == bt, 1.0, 0.0).astype(jnp.float32)
    sblk = jnp.dot(oh, h2, precision=lax.Precision.HIGHEST,
                   preferred_element_type=jnp.float32)
    cblk = jnp.broadcast_to(jnp.sum(oh, axis=1, keepdims=True), (G, D))

    @pl.when(i == 0)
    def _():
        s_acc[...] = sblk
        c_acc[...] = cblk

    @pl.when(i > 0)
    def _():
        s_acc[...] += sblk
        c_acc[...] += cblk

    @pl.when(i == NB - 1)
    def _():
        pooled_ref[...] = s_acc[...] / jnp.maximum(c_acc[...], 1.0)


_tc3 = pl.pallas_call(
    _tc3_body,
    grid=(NB,),
    in_specs=[
        pl.BlockSpec((NC, BN, D), lambda i: (0, i, 0)),
        pl.BlockSpec((BN, D), lambda i: (i, 0)),
        pl.BlockSpec((BN, NW), lambda i: (i, 0)),
        pl.BlockSpec((1, D), lambda i: (0, 0)),
        pl.BlockSpec((1, 1, BN), lambda i: (i, 0, 0)),
    ],
    out_specs=[
        pl.BlockSpec((BN, D), lambda i: (i, 0)),
        pl.BlockSpec((G, D), lambda i: (0, 0)),
    ],
    out_shape=[
        jax.ShapeDtypeStruct((N, D), jnp.float32),
        jax.ShapeDtypeStruct((G, D), jnp.float32),
    ],
    scratch_shapes=[
        pltpu.VMEM((G, D), jnp.float32),
        pltpu.VMEM((G, D), jnp.float32),
    ],
)


def kernel(x, edge_index, batch, W0, b0, W1, b1):
    src3 = edge_index[0].reshape(NW, NSTG, PCH, CH)
    dst3 = edge_index[1].reshape(NW, NSTG, PCH, CH)
    dst1 = edge_index[1]
    zero_rows = jnp.zeros((RCH, D), jnp.float32)

    degp = _deg(dst1)
    deg_t = degp.T

    hs0 = _tc1(deg_t, x, W0)
    acc0 = _agg(hs0, src3, dst3, zero_rows)
    hs1 = _tc2(acc0, hs0, deg_t, b0.reshape(1, D), W1)
    acc1 = _agg(hs1, src3, dst3, zero_rows)
    h2, pooled = _tc3(acc1, hs1, deg_t, b1.reshape(1, D),
                      batch.reshape(NB, 1, BN))
    return (pooled, h2)
